# baseline (speedup 1.0000x reference)
import functools

import jax
import jax.numpy as jnp
from jax import lax
from jax.experimental import pallas as pl
from jax.experimental.pallas import tpu as pltpu
from jax.experimental.pallas import tpu_sc as plsc

B = 16384
NUM_ROWS = 100000
D = 64
DP = 2 * D
NC = 2
NS = 16
NW = NC * NS
BPW = B // NW
CH = 128
NCH = BPW // CH
HALF = BPW // 2
IDX_ROWS = B // CH

_sc_mesh = plsc.VectorSubcoreMesh(core_axis_name="c", subcore_axis_name="s")


@functools.partial(
    pl.kernel,
    mesh=_sc_mesh,
    out_type=jax.ShapeDtypeStruct((B, DP), jnp.float32),
    scratch_types=[
        pltpu.VMEM((NCH, CH), jnp.int32),
        pltpu.VMEM((HALF, DP), jnp.float32),
        pltpu.VMEM((HALF, DP), jnp.float32),
        pltpu.SemaphoreType.DMA,
        pltpu.SemaphoreType.DMA,
    ],
    compiler_params=pltpu.CompilerParams(use_tc_tiling_on_sc=False),
)
def _sc_gather(idx_hbm, cat, out, idx_v, buf0, buf1, sem0, sem1):
    wid = lax.axis_index("s") * NC + lax.axis_index("c")
    base = wid * BPW
    irow = wid * NCH
    pltpu.sync_copy(idx_hbm.at[pl.ds(irow, NCH)], idx_v)

    bufs = (buf0, buf1)
    sems = (sem0, sem1)

    def fire(h):
        buf, sem = bufs[h], sems[h]
        return [
            pltpu.async_copy(cat.at[idx_v.at[2 * h + c]],
                             buf.at[pl.ds(c * CH, CH)], sem)
            for c in range(2)
        ]

    pending = fire(0)
    for h in range(2):
        nxt = fire(h + 1) if h == 0 else None
        for cp in pending:
            cp.wait()
        pltpu.sync_copy(bufs[h], out.at[pl.ds(base + h * HALF, HALF)])
        pending = nxt


PBLK = 12800
NPBLK = (NUM_ROWS + PBLK - 1) // PBLK


def _pack_body(gt, mt, out):
    r = lax.broadcasted_iota(jnp.int32, (D, D), 0)
    c = lax.broadcasted_iota(jnp.int32, (D, D), 1)
    eye = jnp.where(r == c, 1.0, 0.0).astype(jnp.float32)
    dn = (((0,), (0,)), ((), ()))
    out[:, :D] = lax.dot_general(gt[...], eye, dn,
                                 preferred_element_type=jnp.float32)
    out[:, D:] = lax.dot_general(mt[...], eye, dn,
                                 preferred_element_type=jnp.float32)


def _tc_pack(gmf, mlp):
    return pl.pallas_call(
        _pack_body,
        grid=(NPBLK,),
        in_specs=[pl.BlockSpec((D, PBLK), lambda i: (0, i)),
                  pl.BlockSpec((D, PBLK), lambda i: (0, i))],
        out_specs=pl.BlockSpec((PBLK, DP), lambda i: (i, 0)),
        out_shape=jax.ShapeDtypeStruct((NUM_ROWS, DP), jnp.float32),
    )(gmf.T, mlp.T)


BLK = 2048
NBLK = B // BLK


def _tc_body(xu, xi, w1u, w1i, b1, w2, b2, wag, wam, ba, out):
    xu_ = xu[...]
    xi_ = xi[...]
    mu = xu_[:, D:]
    mi = xi_[:, D:]
    h = jnp.dot(mu, w1u[...], preferred_element_type=jnp.float32)
    h = h + jnp.dot(mi, w1i[...], preferred_element_type=jnp.float32)
    h = jnp.maximum(h + b1[...], 0.0)
    h2 = jnp.dot(h, w2[...], preferred_element_type=jnp.float32)
    h2 = jnp.maximum(h2 + b2[...], 0.0)
    g = xu_[:, :D] * xi_[:, :D]
    r = (jnp.sum(g * wag[...], axis=1, keepdims=True)
         + jnp.sum(h2 * wam[...], axis=1, keepdims=True))
    out[...] = r + ba[...]


def _tc_forward(xu, xi, w1u, w1i, b1, w2, b2, wag, wam, ba):
    big = lambda: pl.BlockSpec((BLK, DP), lambda i: (i, 0))
    full = lambda shape: pl.BlockSpec(shape, lambda i: (0, 0))
    return pl.pallas_call(
        _tc_body,
        grid=(NBLK,),
        in_specs=[big(), big(),
                  full((D, 64)), full((D, 64)), full((1, 64)),
                  full((64, 32)), full((1, 32)),
                  full((1, D)), full((1, 32)), full((1, 1))],
        out_specs=pl.BlockSpec((BLK, 1), lambda i: (i, 0)),
        out_shape=jax.ShapeDtypeStruct((B, 1), jnp.float32),
    )(xu, xi, w1u, w1i, b1, w2, b2, wag, wam, ba)


def kernel(user_indices, item_indices, emb_user_gmf, emb_item_gmf,
           emb_user_mlp, emb_item_mlp, W1, b1, W2, b2, Wa, ba):
    ui = jnp.asarray(user_indices, jnp.int32).reshape(IDX_ROWS, CH)
    ii = jnp.asarray(item_indices, jnp.int32).reshape(IDX_ROWS, CH)
    ucat = _tc_pack(emb_user_gmf, emb_user_mlp)
    xu = _sc_gather(ui, ucat)
    icat = _tc_pack(emb_item_gmf, emb_item_mlp)
    xi = _sc_gather(ii, icat)
    w1u, w1i = W1[:D], W1[D:]
    wag = Wa[:D, 0].reshape(1, D)
    wam = Wa[D:, 0].reshape(1, 32)
    return _tc_forward(xu, xi, w1u, w1i, b1.reshape(1, 64),
                       W2, b2.reshape(1, 32), wag, wam, ba.reshape(1, 1))


# speedup vs baseline: 1.0102x; 1.0102x over previous
import functools

import jax
import jax.numpy as jnp
from jax import lax
from jax.experimental import pallas as pl
from jax.experimental.pallas import tpu as pltpu
from jax.experimental.pallas import tpu_sc as plsc

B = 16384
NUM_ROWS = 100000
D = 64
DP = 2 * D
NC = 2
NS = 16
NW = NC * NS
BPW = B // NW
CH = 128
NCH = BPW // CH
HALF = BPW // 2
IDX_ROWS = B // CH

_sc_mesh = plsc.VectorSubcoreMesh(core_axis_name="c", subcore_axis_name="s")


@functools.partial(
    pl.kernel,
    mesh=_sc_mesh,
    out_type=jax.ShapeDtypeStruct((B, DP), jnp.float32),
    scratch_types=[
        pltpu.VMEM((NCH, CH), jnp.int32),
        pltpu.VMEM((HALF, DP), jnp.float32),
        pltpu.VMEM((HALF, DP), jnp.float32),
        pltpu.SemaphoreType.DMA,
        pltpu.SemaphoreType.DMA,
    ],
    compiler_params=pltpu.CompilerParams(use_tc_tiling_on_sc=False),
)
def _sc_gather(idx_hbm, cat, out, idx_v, buf0, buf1, sem0, sem1):
    wid = lax.axis_index("s") * NC + lax.axis_index("c")
    base = wid * BPW
    irow = wid * NCH
    pltpu.sync_copy(idx_hbm.at[pl.ds(irow, NCH)], idx_v)

    bufs = (buf0, buf1)
    sems = (sem0, sem1)

    def fire(h):
        buf, sem = bufs[h], sems[h]
        return [
            pltpu.async_copy(cat.at[idx_v.at[2 * h + c]],
                             buf.at[pl.ds(c * CH, CH)], sem)
            for c in range(2)
        ]

    pending = fire(0)
    for h in range(2):
        nxt = fire(h + 1) if h == 0 else None
        for cp in pending:
            cp.wait()
        pltpu.sync_copy(bufs[h], out.at[pl.ds(base + h * HALF, HALF)])
        pending = nxt


PBLK = 14336
NPBLK = (NUM_ROWS + PBLK - 1) // PBLK


def _pack_body(gt, mt, out):
    out[:, :D] = gt[...].T
    out[:, D:] = mt[...].T


def _tc_pack(gmf, mlp):
    return pl.pallas_call(
        _pack_body,
        grid=(NPBLK,),
        in_specs=[pl.BlockSpec((D, PBLK), lambda i: (0, i)),
                  pl.BlockSpec((D, PBLK), lambda i: (0, i))],
        out_specs=pl.BlockSpec((PBLK, DP), lambda i: (i, 0)),
        out_shape=jax.ShapeDtypeStruct((NUM_ROWS, DP), jnp.float32),
    )(gmf.T, mlp.T)


BLK = 2048
NBLK = B // BLK


def _tc_body(xu, xi, w1u, w1i, b1, w2, b2, wag, wam, ba, out):
    xu_ = xu[...]
    xi_ = xi[...]
    mu = xu_[:, D:]
    mi = xi_[:, D:]
    h = jnp.dot(mu, w1u[...], preferred_element_type=jnp.float32)
    h = h + jnp.dot(mi, w1i[...], preferred_element_type=jnp.float32)
    h = jnp.maximum(h + b1[...], 0.0)
    h2 = jnp.dot(h, w2[...], preferred_element_type=jnp.float32)
    h2 = jnp.maximum(h2 + b2[...], 0.0)
    g = xu_[:, :D] * xi_[:, :D]
    r = (jnp.sum(g * wag[...], axis=1, keepdims=True)
         + jnp.sum(h2 * wam[...], axis=1, keepdims=True))
    out[...] = r + ba[...]


def _tc_forward(xu, xi, w1u, w1i, b1, w2, b2, wag, wam, ba):
    big = lambda: pl.BlockSpec((BLK, DP), lambda i: (i, 0))
    full = lambda shape: pl.BlockSpec(shape, lambda i: (0, 0))
    return pl.pallas_call(
        _tc_body,
        grid=(NBLK,),
        in_specs=[big(), big(),
                  full((D, 64)), full((D, 64)), full((1, 64)),
                  full((64, 32)), full((1, 32)),
                  full((1, D)), full((1, 32)), full((1, 1))],
        out_specs=pl.BlockSpec((BLK, 1), lambda i: (i, 0)),
        out_shape=jax.ShapeDtypeStruct((B, 1), jnp.float32),
    )(xu, xi, w1u, w1i, b1, w2, b2, wag, wam, ba)


def kernel(user_indices, item_indices, emb_user_gmf, emb_item_gmf,
           emb_user_mlp, emb_item_mlp, W1, b1, W2, b2, Wa, ba):
    ui = jnp.asarray(user_indices, jnp.int32).reshape(IDX_ROWS, CH)
    ii = jnp.asarray(item_indices, jnp.int32).reshape(IDX_ROWS, CH)
    ucat = _tc_pack(emb_user_gmf, emb_user_mlp)
    xu = _sc_gather(ui, ucat)
    icat = _tc_pack(emb_item_gmf, emb_item_mlp)
    xi = _sc_gather(ii, icat)
    w1u, w1i = W1[:D], W1[D:]
    wag = Wa[:D, 0].reshape(1, D)
    wam = Wa[D:, 0].reshape(1, 32)
    return _tc_forward(xu, xi, w1u, w1i, b1.reshape(1, 64),
                       W2, b2.reshape(1, 32), wag, wam, ba.reshape(1, 1))
